# baseline (speedup 1.0000x reference)
import jax
import jax.numpy as jnp
from jax.experimental import pallas as pl
from jax.experimental.pallas import tpu as pltpu

N0 = 128
F1 = 8
F2 = 8
D = 6424
H = 256
KT = 292
NSTEP = D // KT


def _fused_body(x0_ref, x1_ref, x2_ref, wbig_ref, wa0_ref,
                ws1_ref, wa1_ref, w1top_ref, b1_ref, lng_ref, lnb_ref,
                w2_ref, b2_ref, out_ref, acc0_ref, acc1_ref):
    k = pl.program_id(0)

    @pl.when(k == 0)
    def _init():
        acc0_ref[...] = jnp.zeros_like(acc0_ref)
        acc1_ref[...] = jnp.zeros_like(acc1_ref)

    xs0 = x0_ref[...].reshape(N0, KT)
    xs1 = x1_ref[...].reshape(N0 * F1, KT)

    wbig = wbig_ref[...].reshape(KT, 2 * H)
    wa0 = wa0_ref[...].reshape(KT, H)
    ws0 = wbig[:, :H]

    m2 = x2_ref[:, 0, 0, 0, :]
    for j in range(1, F2):
        m2 = m2 + x2_ref[:, j, 0, 0, :]
    m2 = m2 * (1.0 / F2)
    m1 = xs1.reshape(N0, F1, KT).sum(axis=1) * (1.0 / F1)

    f32 = jnp.float32
    acc0_ref[...] += (
        jnp.dot(xs0, ws0, preferred_element_type=f32)
        + jnp.dot(m1, wa0, preferred_element_type=f32)
    )
    acc1_ref[...] += jnp.dot(xs1, wbig, preferred_element_type=f32)
    acc1_ref[:, :H] += jnp.dot(m2, wa0, preferred_element_type=f32)

    @pl.when(k == NSTEP - 1)
    def _epilogue():
        h0 = jnp.maximum(acc0_ref[...], 0.0)
        h1 = jnp.maximum(acc1_ref[:, :H], 0.0)
        mh1 = h1.reshape(N0, F1, H).sum(axis=1) * (1.0 / F1)
        g0 = (jnp.dot(h0, ws1_ref[...], preferred_element_type=f32)
              + jnp.dot(mh1, wa1_ref[...], preferred_element_type=f32))
        t = jnp.dot(g0, w1top_ref[...], preferred_element_type=f32)
        trep = jnp.broadcast_to(t[:, None, :], (N0, F1, H)).reshape(N0 * F1, H)
        e = acc1_ref[:, H:] + trep + b1_ref[...]
        mu = e.mean(axis=-1, keepdims=True)
        var = ((e - mu) ** 2).mean(axis=-1, keepdims=True)
        hn = (e - mu) * jax.lax.rsqrt(var + 1e-5) * lng_ref[...] + lnb_ref[...]
        hn = jnp.maximum(hn, 0.0)
        out_ref[...] = (jnp.dot(hn, w2_ref[...], preferred_element_type=f32)
                        + b2_ref[...])


def kernel(x0, x1, x2, W_self0, W_agg0, W_self1, W_agg1,
           mlp_w1, mlp_b1, ln_g, ln_b, mlp_w2, mlp_b2):
    x0v = x0.reshape(N0, NSTEP, 1, KT)
    x1v = x1.reshape(N0 * F1, NSTEP, 1, KT)
    x2v = x2.reshape(N0 * F1, F2, NSTEP, 1, KT)
    wbig = jnp.concatenate([W_self0, mlp_w1[H:]], axis=1)
    wbigv = wbig.reshape(NSTEP, KT, 2 * H)
    wa0v = W_agg0.reshape(NSTEP, KT, H)
    w1top = mlp_w1[:H]
    b1 = mlp_b1.reshape(1, H)
    lng = ln_g.reshape(1, H)
    lnb = ln_b.reshape(1, H)
    b2 = mlp_b2.reshape(1, 1)

    grid = (NSTEP,)
    full = lambda shape: pl.BlockSpec(shape, lambda k: (0,) * len(shape))
    out = pl.pallas_call(
        _fused_body,
        grid=grid,
        in_specs=[
            pl.BlockSpec((N0, 1, 1, KT), lambda k: (0, k, 0, 0)),
            pl.BlockSpec((N0 * F1, 1, 1, KT), lambda k: (0, k, 0, 0)),
            pl.BlockSpec((N0 * F1, F2, 1, 1, KT), lambda k: (0, 0, k, 0, 0)),
            pl.BlockSpec((1, KT, 2 * H), lambda k: (k, 0, 0)),
            pl.BlockSpec((1, KT, H), lambda k: (k, 0, 0)),
            full((H, H)),
            full((H, H)),
            full((H, H)),
            full((1, H)),
            full((1, H)),
            full((1, H)),
            full((H, 1)),
            full((1, 1)),
        ],
        out_specs=pl.BlockSpec((N0 * F1, 1), lambda k: (0, 0)),
        out_shape=jax.ShapeDtypeStruct((N0 * F1, 1), jnp.float32),
        scratch_shapes=[
            pltpu.VMEM((N0, H), jnp.float32),
            pltpu.VMEM((N0 * F1, 2 * H), jnp.float32),
        ],
        compiler_params=pltpu.CompilerParams(
            dimension_semantics=("arbitrary",),
        ),
    )(x0v, x1v, x2v, wbigv, wa0v, W_self1, W_agg1, w1top, b1, lng, lnb,
      mlp_w2, b2)
    return out


# speedup vs baseline: 4.1761x; 4.1761x over previous
import jax
import jax.numpy as jnp
from jax.experimental import pallas as pl
from jax.experimental.pallas import tpu as pltpu

N0 = 128
F1 = 8
F2 = 8
D = 6424
H = 256
B = 8
NSTEP = N0 // B
E = B * F1


def _fused_body(x0_ref, x1_ref, x2_ref, wbig_ref, wa0_ref,
                ws1_ref, wa1_ref, w1top_ref, b1_ref, lng_ref, lnb_ref,
                w2_ref, b2_ref, out_ref):
    f32 = jnp.float32
    x1b = x1_ref[...]
    xs1 = x1b.reshape(E, D)
    m1 = x1b.sum(axis=1) * (1.0 / F1)

    m2 = x2_ref[:, 0, :]
    for j in range(1, F2):
        m2 = m2 + x2_ref[:, j, :]
    m2 = m2 * (1.0 / F2)

    wbig = wbig_ref[...]
    ws0 = wbig[:, :H]
    wa0 = wa0_ref[...]

    h0 = jnp.maximum(
        jnp.dot(x0_ref[...], ws0, preferred_element_type=f32)
        + jnp.dot(m1, wa0, preferred_element_type=f32), 0.0)
    big = jnp.dot(xs1, wbig, preferred_element_type=f32)
    h1 = jnp.maximum(
        big[:, :H] + jnp.dot(m2, wa0, preferred_element_type=f32), 0.0)

    mh1 = h1.reshape(B, F1, H).sum(axis=1) * (1.0 / F1)
    g0 = (jnp.dot(h0, ws1_ref[...], preferred_element_type=f32)
          + jnp.dot(mh1, wa1_ref[...], preferred_element_type=f32))
    t = jnp.dot(g0, w1top_ref[...], preferred_element_type=f32)
    trep = jnp.broadcast_to(t[:, None, :], (B, F1, H)).reshape(E, H)

    e = big[:, H:] + trep + b1_ref[...]
    mu = e.mean(axis=-1, keepdims=True)
    var = ((e - mu) ** 2).mean(axis=-1, keepdims=True)
    hn = (e - mu) * jax.lax.rsqrt(var + 1e-5) * lng_ref[...] + lnb_ref[...]
    hn = jnp.maximum(hn, 0.0)
    out_ref[...] = (jnp.dot(hn, w2_ref[...], preferred_element_type=f32)
                    + b2_ref[...])


def kernel(x0, x1, x2, W_self0, W_agg0, W_self1, W_agg1,
           mlp_w1, mlp_b1, ln_g, ln_b, mlp_w2, mlp_b2):
    x1v = x1.reshape(N0, F1, D)
    x2v = x2.reshape(N0 * F1, F2, D)
    wbig = jnp.concatenate([W_self0, mlp_w1[H:]], axis=1)
    w1top = mlp_w1[:H]
    b1 = mlp_b1.reshape(1, H)
    lng = ln_g.reshape(1, H)
    lnb = ln_b.reshape(1, H)
    b2 = mlp_b2.reshape(1, 1)

    full = lambda shape: pl.BlockSpec(shape, lambda i: (0,) * len(shape))
    out = pl.pallas_call(
        _fused_body,
        grid=(NSTEP,),
        in_specs=[
            pl.BlockSpec((B, D), lambda i: (i, 0)),
            pl.BlockSpec((B, F1, D), lambda i: (i, 0, 0)),
            pl.BlockSpec((E, F2, D), lambda i: (i, 0, 0)),
            full((D, 2 * H)),
            full((D, H)),
            full((H, H)),
            full((H, H)),
            full((H, H)),
            full((1, H)),
            full((1, H)),
            full((1, H)),
            full((H, 1)),
            full((1, 1)),
        ],
        out_specs=pl.BlockSpec((E, 1), lambda i: (i, 0)),
        out_shape=jax.ShapeDtypeStruct((N0 * F1, 1), jnp.float32),
        compiler_params=pltpu.CompilerParams(
            dimension_semantics=("arbitrary",),
        ),
    )(x0, x1v, x2v, wbig, W_agg0, W_self1, W_agg1, w1top, b1, lng, lnb,
      mlp_w2, b2)
    return out
